# Initial kernel scaffold; baseline (speedup 1.0000x reference)
import jax
import jax.numpy as jnp
from jax.experimental import pallas as pl


def kernel(x, values, mix_w, mix_b):
    raise NotImplementedError("write your pallas kernel here")


# speedup vs baseline: 34.4305x; 34.4305x over previous
import numpy as np
import jax
import jax.numpy as jnp
from jax.experimental import pallas as pl

IN_DIM_K = 8192
OUT_DIM_K = 16
GRID_K = 16

_KNOTS = np.linspace(-1.0, 1.0, GRID_K).astype(np.float32)
_INV = (1.0 / (_KNOTS[1:] - _KNOTS[:-1] + np.float32(1e-8))).astype(np.float32)


def _fused_kernel(x_ref, vt_ref, mwt_ref, b_ref, out_ref):
    xc = jnp.clip(x_ref[...], -1.0, 1.0)

    idxf = jnp.full_like(xc, 1.0)
    x0 = jnp.full_like(xc, _KNOTS[0])
    invd = jnp.full_like(xc, _INV[0])
    for g in range(1, GRID_K - 1):
        c = (xc > _KNOTS[g]).astype(jnp.float32)
        idxf = idxf + c
        x0 = x0 + c * (_KNOTS[g] - _KNOTS[g - 1])
        invd = invd + c * (_INV[g] - _INV[g - 1])
    t = (xc - x0) * invd

    vt = vt_ref[...]
    mwt = mwt_ref[...]
    tv = vt * t
    mv = jnp.dot(vt, mwt, preferred_element_type=jnp.float32)
    mt = jnp.dot(tv, mwt, preferred_element_type=jnp.float32)

    a_hi = mv[: GRID_K - 1, :] + mt[1:, :] - mt[: GRID_K - 1, :] + b_ref[...]
    a = jnp.concatenate([jnp.zeros((1, OUT_DIM_K), jnp.float32), a_hi], axis=0)

    kcol = jax.lax.broadcasted_iota(jnp.int32, (GRID_K, IN_DIM_K), 0)
    ot = (kcol == idxf.astype(jnp.int32)).astype(jnp.float32)
    out_ref[...] = jax.lax.dot_general(
        ot, a, dimension_numbers=(((0,), (0,)), ((), ())),
        preferred_element_type=jnp.float32,
    )


def kernel(x, values, mix_w, mix_b):
    xr = x.reshape(1, IN_DIM_K)
    vt = values.T
    mwt = mix_w.T
    br = mix_b.reshape(1, OUT_DIM_K)
    return pl.pallas_call(
        _fused_kernel,
        out_shape=jax.ShapeDtypeStruct((IN_DIM_K, OUT_DIM_K), jnp.float32),
    )(xr, vt, mwt, br)
